# baseline (speedup 1.0000x reference)
import functools

import jax
import jax.numpy as jnp
from jax import lax
from jax.experimental import pallas as pl
from jax.experimental.pallas import tpu as pltpu
from jax.experimental.pallas import tpu_sc as plsc

_VOCAB = 1000000
_HIDDEN = 64
_BATCH = 4096
_SEQ = 200

_TOT = _BATCH * _SEQ
_CH = 128
_NBC = _BATCH // _CH
_NC = 2
_NS = 16
_NW = _NC * _NS
_CPW = _TOT // (_CH * _NW)
_L = 16

_mesh = plsc.VectorSubcoreMesh(core_axis_name="c", subcore_axis_name="s")


@functools.partial(
    pl.kernel,
    out_type=jax.ShapeDtypeStruct((_SEQ, _HIDDEN // 8, _NBC, 8 * _CH), jnp.float32),
    mesh=_mesh,
    compiler_params=pltpu.CompilerParams(
        use_tc_tiling_on_sc=False, needs_layout_passes=False
    ),
    scratch_types=[
        pltpu.VMEM((_CPW, _CH), jnp.int32),
        pltpu.VMEM((_CH, _HIDDEN), jnp.float32),
        pltpu.VMEM((_CH, _HIDDEN), jnp.float32),
        pltpu.VMEM((_CH * _HIDDEN,), jnp.float32),
        pltpu.VMEM((_CH * _HIDDEN,), jnp.float32),
        pltpu.SemaphoreType.DMA,
        pltpu.SemaphoreType.DMA,
        pltpu.SemaphoreType.DMA,
        pltpu.SemaphoreType.DMA,
    ],
)
def _embed_gather(ids_hbm, table_hbm, out_hbm, idx_v, buf_a, buf_b, tb_a, tb_b,
                  gsem_a, gsem_b, psem_a, psem_b):
    wid = lax.axis_index("s") * _NC + lax.axis_index("c")
    crow = wid * _CPW
    pltpu.sync_copy(ids_hbm.at[pl.ds(crow, _CPW)], idx_v)

    iota = lax.iota(jnp.int32, _L)
    idx_dst = [(iota + _L * m) * _CH for m in range(_HIDDEN // _L)]

    def gather(c, buf, gsem):
        return pltpu.make_async_copy(table_hbm.at[idx_v.at[c]], buf, gsem)

    def drain(buf, sem):
        pltpu.make_async_copy(table_hbm.at[pl.ds(0, _CH)], buf, sem).wait()

    def transpose(buf, tb):
        for r in range(_CH):
            for m in range(_HIDDEN // _L):
                plsc.store_scatter(tb, [idx_dst[m] + r], buf[r, pl.ds(_L * m, _L)])

    def put(c, tb, psem):
        s = (crow + c) // _NBC
        bc = lax.rem(crow + c, _NBC)
        for jr in range(_HIDDEN // 8):
            pltpu.make_async_copy(
                tb.at[pl.ds(jr * 8 * _CH, 8 * _CH)], out_hbm.at[s, jr, bc], psem
            ).start()

    gather(0, buf_a, gsem_a).start()

    def step(t, carry):
        c0 = 2 * t
        c1 = c0 + 1

        gather(c1, buf_b, gsem_b).start()
        drain(buf_a, gsem_a)

        @pl.when(t > 0)
        def _():
            drain(buf_a, psem_a)

        transpose(buf_a, tb_a)
        put(c0, tb_a, psem_a)

        @pl.when(t < _CPW // 2 - 1)
        def _():
            gather(c0 + 2, buf_a, gsem_a).start()

        drain(buf_b, gsem_b)

        @pl.when(t > 0)
        def _():
            drain(buf_b, psem_b)

        transpose(buf_b, tb_b)
        put(c1, tb_b, psem_b)
        return carry

    lax.fori_loop(0, _CPW // 2, step, 0)
    drain(buf_a, psem_a)
    drain(buf_b, psem_b)


def kernel(input_ids, token_embeddings):
    ids2 = input_ids.T.reshape(_TOT // _CH, _CH)
    r = _embed_gather(ids2, token_embeddings)
    r = r.reshape(_SEQ, _HIDDEN // 8, _NBC, 8, _CH)
    return r.transpose(2, 4, 0, 1, 3).reshape(_BATCH, _SEQ, _HIDDEN)


# speedup vs baseline: 1.2300x; 1.2300x over previous
import functools

import jax
import jax.numpy as jnp
from jax import lax
from jax.experimental import pallas as pl
from jax.experimental.pallas import tpu as pltpu
from jax.experimental.pallas import tpu_sc as plsc

_VOCAB = 1000000
_HIDDEN = 64
_BATCH = 4096
_SEQ = 200

_TOT = _BATCH * _SEQ
_CH = 128
_NBC = _BATCH // _CH
_NC = 2
_NS = 16
_NW = _NC * _NS
_CPW = _TOT // (_CH * _NW)
_L = 16

_mesh = plsc.VectorSubcoreMesh(core_axis_name="c", subcore_axis_name="s")


@functools.partial(
    pl.kernel,
    out_type=jax.ShapeDtypeStruct((_SEQ, _HIDDEN // 8, _NBC, 8 * _CH), jnp.float32),
    mesh=_mesh,
    compiler_params=pltpu.CompilerParams(
        use_tc_tiling_on_sc=False, needs_layout_passes=False
    ),
    scratch_types=[
        pltpu.VMEM((_CPW, _CH), jnp.int32),
        pltpu.VMEM((_CH, _HIDDEN), jnp.float32),
        pltpu.VMEM((_CH, _HIDDEN), jnp.float32),
        pltpu.VMEM((_CH * _HIDDEN,), jnp.float32),
        pltpu.VMEM((_CH * _HIDDEN,), jnp.float32),
        pltpu.SemaphoreType.DMA,
        pltpu.SemaphoreType.DMA,
        pltpu.SemaphoreType.DMA,
        pltpu.SemaphoreType.DMA,
    ],
)
def _embed_gather(ids_hbm, table_hbm, out_hbm, idx_v, buf_a, buf_b, tb_a, tb_b,
                  gsem_a, gsem_b, psem_a, psem_b):
    wid = lax.axis_index("s") * _NC + lax.axis_index("c")
    crow = wid * _CPW
    pltpu.sync_copy(ids_hbm.at[pl.ds(crow, _CPW)], idx_v)

    iota = lax.iota(jnp.int32, _L)
    idx_dst = [(iota + _L * m) * _CH for m in range(_HIDDEN // _L)]

    def gather(c, buf, gsem):
        return pltpu.make_async_copy(table_hbm.at[idx_v.at[c]], buf, gsem)

    def drain(buf, sem):
        pltpu.make_async_copy(table_hbm.at[pl.ds(0, _CH)], buf, sem).wait()

    def transpose(buf, tb):
        @plsc.parallel_loop(0, _CH, unroll=8)
        def _(r):
            for m in range(_HIDDEN // _L):
                plsc.store_scatter(tb, [idx_dst[m] + r], buf[r, pl.ds(_L * m, _L)])

    def put(c, tb, psem):
        s = (crow + c) // _NBC
        bc = lax.rem(crow + c, _NBC)
        for jr in range(_HIDDEN // 8):
            pltpu.make_async_copy(
                tb.at[pl.ds(jr * 8 * _CH, 8 * _CH)], out_hbm.at[s, jr, bc], psem
            ).start()

    gather(0, buf_a, gsem_a).start()

    def step(t, carry):
        c0 = 2 * t
        c1 = c0 + 1

        gather(c1, buf_b, gsem_b).start()
        drain(buf_a, gsem_a)

        @pl.when(t > 0)
        def _():
            drain(buf_a, psem_a)

        transpose(buf_a, tb_a)
        put(c0, tb_a, psem_a)

        @pl.when(t < _CPW // 2 - 1)
        def _():
            gather(c0 + 2, buf_a, gsem_a).start()

        drain(buf_b, gsem_b)

        @pl.when(t > 0)
        def _():
            drain(buf_b, psem_b)

        transpose(buf_b, tb_b)
        put(c1, tb_b, psem_b)
        return carry

    lax.fori_loop(0, _CPW // 2, step, 0)
    drain(buf_a, psem_a)
    drain(buf_b, psem_b)


def kernel(input_ids, token_embeddings):
    ids2 = input_ids.T.reshape(_TOT // _CH, _CH)
    r = _embed_gather(ids2, token_embeddings)
    r = r.reshape(_SEQ, _HIDDEN // 8, _NBC, 8, _CH)
    return r.transpose(2, 4, 0, 1, 3).reshape(_BATCH, _SEQ, _HIDDEN)
